# baseline (speedup 1.0000x reference)
import jax
import jax.numpy as jnp
from jax import lax
from jax.experimental import pallas as pl
from jax.experimental.pallas import tpu as pltpu
from jax.experimental.pallas import tpu_sc as plsc

MAX_LEN = 2048
D_MODEL = 4096
BATCH = 4
NC = 2
NS = 16
NW = NC * NS
POS_PER_W = MAX_LEN // NW
LANES = 16
UNROLL = 8
STEPS = D_MODEL // (LANES * UNROLL)


def _body(in_hbm, tab_hbm, out_hbm, tab_v, row_v):
    wid = lax.axis_index("s") * NC + lax.axis_index("c")

    def pos_step(i, carry):
        l = wid * POS_PER_W + i
        pltpu.sync_copy(tab_hbm.at[l], tab_v)
        for b in range(BATCH):
            pltpu.sync_copy(in_hbm.at[b, l], row_v)

            def chunk_step(c, carry2):
                base = c * (LANES * UNROLL)
                for u in range(UNROLL):
                    o = base + u * LANES
                    row_v[pl.ds(o, LANES)] = (
                        row_v[pl.ds(o, LANES)] + tab_v[pl.ds(o, LANES)]
                    )
                return carry2

            lax.fori_loop(0, STEPS, chunk_step, 0)
            pltpu.sync_copy(row_v, out_hbm.at[b, l])
        return carry

    lax.fori_loop(0, POS_PER_W, pos_step, 0)


def kernel(inputs, pos_table):
    k = pl.kernel(
        _body,
        out_type=jax.ShapeDtypeStruct((BATCH, MAX_LEN, D_MODEL), jnp.float32),
        mesh=plsc.VectorSubcoreMesh(core_axis_name="c", subcore_axis_name="s"),
        scratch_types=[
            pltpu.VMEM((D_MODEL,), jnp.float32),
            pltpu.VMEM((D_MODEL,), jnp.float32),
        ],
    )
    return k(inputs, pos_table)


# speedup vs baseline: 1.0231x; 1.0231x over previous
import jax
import jax.numpy as jnp
from jax import lax
from jax.experimental import pallas as pl
from jax.experimental.pallas import tpu as pltpu
from jax.experimental.pallas import tpu_sc as plsc

MAX_LEN = 2048
D_MODEL = 4096
BATCH = 4
NC = 2
NS = 16
NW = NC * NS
ROWS_PER_W = (MAX_LEN * BATCH) // NW
LANES = 16
T = 2
GELEMS = T * D_MODEL
NG = ROWS_PER_W // T
NPHASE = 4
UNROLL = 8
CSTEPS = GELEMS // (LANES * UNROLL)


def _body(in_hbm, tab_hbm, out_hbm, *scratch):
    rb = scratch[0:NPHASE]
    tb = scratch[NPHASE:2 * NPHASE]
    si = scratch[2 * NPHASE:3 * NPHASE]
    st = scratch[3 * NPHASE:4 * NPHASE]
    so = scratch[4 * NPHASE:5 * NPHASE]

    wid = lax.axis_index("s") * NC + lax.axis_index("c")
    in_base = wid * (ROWS_PER_W * D_MODEL)
    tab_base = lax.rem(wid, NW // BATCH) * (ROWS_PER_W * D_MODEL)

    def start_reads(g, p):
        pltpu.async_copy(in_hbm.at[pl.ds(in_base + g * GELEMS, GELEMS)],
                         rb[p], si[p])
        pltpu.async_copy(tab_hbm.at[pl.ds(tab_base + g * GELEMS, GELEMS)],
                         tb[p], st[p])

    def wait_read(p):
        pltpu.make_async_copy(in_hbm.at[pl.ds(0, GELEMS)], rb[p], si[p]).wait()
        pltpu.make_async_copy(tab_hbm.at[pl.ds(0, GELEMS)], tb[p], st[p]).wait()

    def wait_write(p):
        pltpu.make_async_copy(rb[p], out_hbm.at[pl.ds(0, GELEMS)], so[p]).wait()

    for g in range(2):
        start_reads(g, g)

    def group_body(gg, carry):
        for p in range(NPHASE):
            g = gg * NPHASE + p
            pn = (p + 2) % NPHASE

            @pl.when(g + 2 < NG)
            def _():
                @pl.when(g >= 2)
                def _():
                    wait_write(pn)
                start_reads(g + 2, pn)

            wait_read(p)

            def chunk_step(c, carry2):
                base = c * (LANES * UNROLL)
                for u in range(UNROLL):
                    o = base + u * LANES
                    rb[p][pl.ds(o, LANES)] = (
                        rb[p][pl.ds(o, LANES)] + tb[p][pl.ds(o, LANES)]
                    )
                return carry2

            lax.fori_loop(0, CSTEPS, chunk_step, 0)

            pltpu.async_copy(rb[p], out_hbm.at[pl.ds(in_base + g * GELEMS,
                                                     GELEMS)], so[p])
        return carry

    lax.fori_loop(0, NG // NPHASE, group_body, 0)

    for p in range(NPHASE):
        wait_write(p)


def kernel(inputs, pos_table):
    k = pl.kernel(
        _body,
        out_type=jax.ShapeDtypeStruct((MAX_LEN * BATCH * D_MODEL,),
                                      jnp.float32),
        mesh=plsc.VectorSubcoreMesh(core_axis_name="c", subcore_axis_name="s"),
        scratch_types=(
            [pltpu.VMEM((GELEMS,), jnp.float32) for _ in range(NPHASE)]
            + [pltpu.VMEM((GELEMS,), jnp.float32) for _ in range(NPHASE)]
            + [pltpu.SemaphoreType.DMA for _ in range(3 * NPHASE)]
        ),
    )
    out = k(inputs.reshape(-1), pos_table.reshape(-1))
    return out.reshape(BATCH, MAX_LEN, D_MODEL)


# speedup vs baseline: 2.6911x; 2.6303x over previous
import jax
import jax.numpy as jnp
from jax import lax
from jax.experimental import pallas as pl
from jax.experimental.pallas import tpu as pltpu
from jax.experimental.pallas import tpu_sc as plsc

MAX_LEN = 2048
D_MODEL = 4096
BATCH = 4
NC = 2
NS = 16
NW = NC * NS
ROWS_PER_W = (MAX_LEN * BATCH) // NW
LANES = 16
T = 2
GELEMS = T * D_MODEL
NG = ROWS_PER_W // T
NPHASE = 4
UNROLL = 8
CSTEPS = D_MODEL // (LANES * UNROLL)


def _body(in_hbm, tab_hbm, out_hbm, *scratch):
    rb = scratch[0:NPHASE]
    tb = scratch[NPHASE:2 * NPHASE]
    si = scratch[2 * NPHASE:3 * NPHASE]
    st = scratch[3 * NPHASE:4 * NPHASE]
    so = scratch[4 * NPHASE:5 * NPHASE]

    wid = lax.axis_index("s") * NC + lax.axis_index("c")
    batch = wid // (NW // BATCH)
    l_base = lax.rem(wid, NW // BATCH) * ROWS_PER_W

    def start_reads(g, p):
        pltpu.async_copy(in_hbm.at[batch, pl.ds(l_base + g * T, T)],
                         rb[p], si[p])
        pltpu.async_copy(tab_hbm.at[pl.ds(l_base + g * T, T)],
                         tb[p], st[p])

    def wait_read(p):
        pltpu.make_async_copy(in_hbm.at[0, pl.ds(0, T)], rb[p], si[p]).wait()
        pltpu.make_async_copy(tab_hbm.at[pl.ds(0, T)], tb[p], st[p]).wait()

    def wait_write(p):
        pltpu.make_async_copy(rb[p], out_hbm.at[0, pl.ds(0, T)], so[p]).wait()

    for g in range(2):
        start_reads(g, g)

    def group_body(gg, carry):
        for p in range(NPHASE):
            g = gg * NPHASE + p
            pn = (p + 2) % NPHASE

            @pl.when(g + 2 < NG)
            def _():
                @pl.when(g >= 2)
                def _():
                    wait_write(pn)
                start_reads(g + 2, pn)

            wait_read(p)

            for r in range(T):
                def chunk_step(c, carry2, _r=r):
                    base = c * (LANES * UNROLL)
                    for u in range(UNROLL):
                        o = base + u * LANES
                        rb[p][_r, pl.ds(o, LANES)] = (
                            rb[p][_r, pl.ds(o, LANES)]
                            + tb[p][_r, pl.ds(o, LANES)]
                        )
                    return carry2

                lax.fori_loop(0, CSTEPS, chunk_step, 0)

            pltpu.async_copy(rb[p],
                             out_hbm.at[batch, pl.ds(l_base + g * T, T)],
                             so[p])
        return carry

    lax.fori_loop(0, NG // NPHASE, group_body, 0)

    for p in range(NPHASE):
        wait_write(p)


def kernel(inputs, pos_table):
    k = pl.kernel(
        _body,
        out_type=jax.ShapeDtypeStruct((BATCH, MAX_LEN, D_MODEL), jnp.float32),
        mesh=plsc.VectorSubcoreMesh(core_axis_name="c", subcore_axis_name="s"),
        scratch_types=(
            [pltpu.VMEM((T, D_MODEL), jnp.float32) for _ in range(NPHASE)]
            + [pltpu.VMEM((T, D_MODEL), jnp.float32) for _ in range(NPHASE)]
            + [pltpu.SemaphoreType.DMA for _ in range(3 * NPHASE)]
        ),
    )
    return k(inputs, pos_table)


# speedup vs baseline: 3.4977x; 1.2997x over previous
import jax
import jax.numpy as jnp
from jax import lax
from jax.experimental import pallas as pl
from jax.experimental.pallas import tpu as pltpu
from jax.experimental.pallas import tpu_sc as plsc

MAX_LEN = 2048
D_MODEL = 4096
BATCH = 4
NC = 2
NS = 16
NW = NC * NS
POS_PER_W = MAX_LEN // NW
LANES = 16
NCHUNK = D_MODEL // LANES
NPHASE = 4
UNROLL = 4


def _body(in_hbm, tab_hbm, out_hbm, *scratch):
    rb = scratch[0:NPHASE]
    tb = scratch[NPHASE:2 * NPHASE]
    si = scratch[2 * NPHASE:3 * NPHASE]
    st = scratch[3 * NPHASE:4 * NPHASE]
    so = scratch[4 * NPHASE:5 * NPHASE]

    wid = lax.axis_index("s") * NC + lax.axis_index("c")
    l_base = wid * POS_PER_W

    def start_reads(g, p):
        l = l_base + g
        pltpu.async_copy(in_hbm.at[:, l], rb[p], si[p])
        pltpu.async_copy(tab_hbm.at[l], tb[p], st[p])

    def wait_read(p):
        pltpu.make_async_copy(in_hbm.at[:, 0], rb[p], si[p]).wait()
        pltpu.make_async_copy(tab_hbm.at[0], tb[p], st[p]).wait()

    def wait_write(p):
        pltpu.make_async_copy(rb[p], out_hbm.at[:, 0], so[p]).wait()

    for g in range(2):
        start_reads(g, g)

    def group_body(gg, carry):
        for p in range(NPHASE):
            g = gg * NPHASE + p
            pn = (p + 2) % NPHASE

            @pl.when(g + 2 < POS_PER_W)
            def _():
                @pl.when(g >= 2)
                def _():
                    wait_write(pn)
                start_reads(g + 2, pn)

            wait_read(p)

            @plsc.parallel_loop(0, NCHUNK, 1, unroll=UNROLL)
            def chunk_step(c, _p=p):
                o = c * LANES
                t = tb[_p][pl.ds(o, LANES)]
                for b in range(BATCH):
                    plsc.addupdate(rb[_p].at[b, pl.ds(o, LANES)], t)

            pltpu.async_copy(rb[p], out_hbm.at[:, l_base + g], so[p])
        return carry

    lax.fori_loop(0, POS_PER_W // NPHASE, group_body, 0)

    for p in range(NPHASE):
        wait_write(p)


def kernel(inputs, pos_table):
    k = pl.kernel(
        _body,
        out_type=jax.ShapeDtypeStruct((BATCH, MAX_LEN, D_MODEL), jnp.float32),
        mesh=plsc.VectorSubcoreMesh(core_axis_name="c", subcore_axis_name="s"),
        scratch_types=(
            [pltpu.VMEM((BATCH, D_MODEL), jnp.float32) for _ in range(NPHASE)]
            + [pltpu.VMEM((D_MODEL,), jnp.float32) for _ in range(NPHASE)]
            + [pltpu.SemaphoreType.DMA for _ in range(3 * NPHASE)]
        ),
    )
    return k(inputs, pos_table)
